# baseline (speedup 1.0000x reference)
import functools

import jax
import jax.numpy as jnp
from jax import lax
from jax.experimental import pallas as pl
from jax.experimental.pallas import tpu as pltpu
from jax.experimental.pallas import tpu_sc as plsc

DIM = 64
NEG = 10
L = 16
C = 64


def _sc_scores(x, targets, neg_flat, emb2, w2):
    B = x.shape[0]
    info = plsc.get_sparse_core_info()
    NC, NS = info.num_cores, info.num_subcores
    NW = NC * NS
    per_w = B // NW
    n_chunks = per_w // C
    NCH = C * NEG // 128

    mesh = plsc.VectorSubcoreMesh(core_axis_name="c", subcore_axis_name="s")

    @functools.partial(
        pl.kernel,
        mesh=mesh,
        compiler_params=pltpu.CompilerParams(
            needs_layout_passes=False, use_tc_tiling_on_sc=True
        ),
        out_type=(
            jax.ShapeDtypeStruct((B,), jnp.float32),
            jax.ShapeDtypeStruct((B * NEG,), jnp.float32),
        ),
        scratch_types=[
            pltpu.VMEM((C,), jnp.int32),
            pltpu.VMEM((C,), jnp.int32),
            pltpu.VMEM((C * NEG,), jnp.int32),
            pltpu.VMEM((C,), jnp.int32),
            pltpu.VMEM((C,), jnp.int32),
            pltpu.VMEM((NCH, 128), jnp.int32),
            pltpu.VMEM((C, 2 * DIM), jnp.float32),
            pltpu.VMEM((C, 2 * DIM), jnp.float32),
            pltpu.VMEM((C * NEG, 2 * DIM), jnp.float32),
            pltpu.VMEM((C,), jnp.float32),
            pltpu.VMEM((C * NEG,), jnp.float32),
            pltpu.SemaphoreType.DMA,
        ],
    )
    def k(x_h, t_h, n_h, emb_h, w_h, pos_h, negs_h,
          re, rp, rn, pe, pp, pn, emb_v, pos_v, neg_v, pos_o, neg_o, sem):
        wid = lax.axis_index("s") * NC + lax.axis_index("c")
        base = wid * per_w

        def chunk_body(ci, _):
            cbase = base + ci * C
            pltpu.sync_copy(x_h.at[pl.ds(cbase, C)], re)
            pltpu.sync_copy(t_h.at[pl.ds(cbase, C)], rp)
            pltpu.sync_copy(n_h.at[pl.ds(cbase * NEG, C * NEG)], rn)
            for v in range(C // L):
                pe[pl.ds(v * L, L)] = re[pl.ds(v * L, L)] >> 1
                pp[pl.ds(v * L, L)] = rp[pl.ds(v * L, L)] >> 1
            for m in range(NCH):
                for v in range(128 // L):
                    pn[m, pl.ds(v * L, L)] = rn[pl.ds(m * 128 + v * L, L)] >> 1
            copies = [
                pltpu.async_copy(emb_h.at[pe], emb_v, sem),
                pltpu.async_copy(w_h.at[pp], pos_v, sem),
            ]
            for m in range(NCH):
                copies.append(
                    pltpu.async_copy(
                        w_h.at[pn.at[m]], neg_v.at[pl.ds(m * 128, 128)], sem
                    )
                )
            for cp in copies:
                cp.wait()

            def body(g, _):
                rows = g * L + lax.iota(jnp.int32, L)
                ce = (re[pl.ds(g * L, L)] & 1) * DIM
                cp_ = (rp[pl.ds(g * L, L)] & 1) * DIM
                nrows = [rows * NEG + j for j in range(NEG)]
                cn = [(plsc.load_gather(rn, [nrows[j]]) & 1) * DIM
                      for j in range(NEG)]
                pos_acc = jnp.zeros((L,), jnp.float32)
                neg_accs = [jnp.zeros((L,), jnp.float32) for _ in range(NEG)]
                for d in range(DIM):
                    ev = plsc.load_gather(emb_v, [rows, ce + d])
                    pv = plsc.load_gather(pos_v, [rows, cp_ + d])
                    pos_acc = pos_acc + ev * pv
                    for j in range(NEG):
                        nv = plsc.load_gather(neg_v, [nrows[j], cn[j] + d])
                        neg_accs[j] = neg_accs[j] + ev * nv
                pos_o[pl.ds(g * L, L)] = pos_acc
                for j in range(NEG):
                    neg_o[pl.ds(j * C + g * L, L)] = neg_accs[j]
                return 0

            lax.fori_loop(0, C // L, body, 0)
            pltpu.sync_copy(pos_o, pos_h.at[pl.ds(cbase, C)])
            pltpu.sync_copy(neg_o, negs_h.at[pl.ds(cbase * NEG, C * NEG)])
            return 0

        lax.fori_loop(0, n_chunks, chunk_body, 0)

    return k(x, targets, neg_flat, emb2, w2)


def _tc_loss(pos, neg, B):
    def body(pos_ref, neg_ref, out_ref):
        p = pos_ref[...]
        n = neg_ref[...]
        sp = jnp.maximum(-p, 0.0) + jnp.log1p(jnp.exp(-jnp.abs(p)))
        sn = jnp.maximum(n, 0.0) + jnp.log1p(jnp.exp(-jnp.abs(n)))
        out_ref[...] = ((jnp.sum(sp) + jnp.sum(sn)) * (1.0 / B)).reshape(1, 1)

    res = pl.pallas_call(
        body,
        out_shape=jax.ShapeDtypeStruct((1, 1), jnp.float32),
    )(pos.reshape(B // 128, 128), neg.reshape(B * NEG // 128, 128))
    return res[0, 0]


def kernel(x, targets, negatives, emb_table, out_weight):
    B = x.shape[0]
    V = emb_table.shape[0]
    x = x.astype(jnp.int32)
    targets = targets.astype(jnp.int32)
    neg_flat = negatives.astype(jnp.int32).reshape(-1)
    emb2 = emb_table.reshape(V // 2, 2 * DIM)
    w2 = out_weight.reshape(V // 2, 2 * DIM)
    pos_s, neg_s = _sc_scores(x, targets, neg_flat, emb2, w2)
    return _tc_loss(pos_s, neg_s, B)


# speedup vs baseline: 1.4064x; 1.4064x over previous
import functools

import jax
import jax.numpy as jnp
from jax import lax
from jax.experimental import pallas as pl
from jax.experimental.pallas import tpu as pltpu
from jax.experimental.pallas import tpu_sc as plsc

DIM = 64
NEG = 10
L = 16
C = 64


def _sc_scores(x, targets, neg_flat, emb_table, out_weight):
    B = x.shape[0]
    info = plsc.get_sparse_core_info()
    NC, NS = info.num_cores, info.num_subcores
    NW = NC * NS
    per_w = B // NW
    n_chunks = per_w // C

    mesh = plsc.VectorSubcoreMesh(core_axis_name="c", subcore_axis_name="s")

    @functools.partial(
        pl.kernel,
        mesh=mesh,
        compiler_params=pltpu.CompilerParams(
            needs_layout_passes=False, use_tc_tiling_on_sc=True
        ),
        out_type=(
            jax.ShapeDtypeStruct((B,), jnp.float32),
            jax.ShapeDtypeStruct((B * NEG,), jnp.float32),
        ),
        scratch_types=[
            pltpu.VMEM((C,), jnp.int32),
            pltpu.VMEM((C,), jnp.int32),
            pltpu.VMEM((C * NEG,), jnp.int32),
            pltpu.VMEM((C, DIM), jnp.float32),
            pltpu.VMEM((C, DIM), jnp.float32),
            pltpu.VMEM((C * NEG, DIM), jnp.float32),
            pltpu.VMEM((C,), jnp.float32),
            pltpu.VMEM((C * NEG,), jnp.float32),
            pltpu.SemaphoreType.DMA,
        ],
    )
    def k(x_h, t_h, n_h, emb_h, w_h, pos_h, negs_h,
          vi_x, vi_t, vi_n, emb_v, pos_v, neg_v,
          pos_o, neg_o, sem):
        wid = lax.axis_index("s") * NC + lax.axis_index("c")
        base = wid * per_w

        def chunk_body(ci, _):
            cbase = base + ci * C
            pltpu.sync_copy(x_h.at[pl.ds(cbase, C)], vi_x)
            pltpu.sync_copy(t_h.at[pl.ds(cbase, C)], vi_t)
            pltpu.sync_copy(n_h.at[pl.ds(cbase * NEG, C * NEG)], vi_n)

            def row_issue(g, _):
                vx = vi_x[pl.ds(g * L, L)]
                vt = vi_t[pl.ds(g * L, L)]
                for t in range(L):
                    i = g * L + t
                    pltpu.async_copy(emb_h.at[vx[t]], emb_v.at[i], sem)
                    pltpu.async_copy(w_h.at[vt[t]], pos_v.at[i], sem)
                return 0

            def neg_issue(g, _):
                vn = vi_n[pl.ds(g * L, L)]
                for t in range(L):
                    i = g * L + t
                    pltpu.async_copy(w_h.at[vn[t]], neg_v.at[i], sem)
                return 0

            lax.fori_loop(0, C // L, row_issue, 0)
            lax.fori_loop(0, C * NEG // L, neg_issue, 0)
            pltpu.make_async_copy(emb_h.at[pl.ds(0, C)], emb_v, sem).wait()
            pltpu.make_async_copy(w_h.at[pl.ds(0, C)], pos_v, sem).wait()
            pltpu.make_async_copy(w_h.at[pl.ds(0, C * NEG)], neg_v, sem).wait()

            def body(g, _):
                rows = g * L + lax.iota(jnp.int32, L)
                nrows = [rows * NEG + j for j in range(NEG)]
                pos_acc = jnp.zeros((L,), jnp.float32)
                neg_accs = [jnp.zeros((L,), jnp.float32) for _ in range(NEG)]
                for d in range(DIM):
                    cold = jnp.full((L,), d, jnp.int32)
                    ev = plsc.load_gather(emb_v, [rows, cold])
                    pv = plsc.load_gather(pos_v, [rows, cold])
                    pos_acc = pos_acc + ev * pv
                    for j in range(NEG):
                        nv = plsc.load_gather(neg_v, [nrows[j], cold])
                        neg_accs[j] = neg_accs[j] + ev * nv
                pos_o[pl.ds(g * L, L)] = pos_acc
                for j in range(NEG):
                    neg_o[pl.ds(j * C + g * L, L)] = neg_accs[j]
                return 0

            lax.fori_loop(0, C // L, body, 0)
            pltpu.sync_copy(pos_o, pos_h.at[pl.ds(cbase, C)])
            pltpu.sync_copy(neg_o, negs_h.at[pl.ds(cbase * NEG, C * NEG)])
            return 0

        lax.fori_loop(0, n_chunks, chunk_body, 0)

    return k(x, targets, neg_flat, emb_table, out_weight)


def _tc_loss(pos, neg, B):
    def body(pos_ref, neg_ref, out_ref):
        p = pos_ref[...]
        n = neg_ref[...]
        sp = jnp.maximum(-p, 0.0) + jnp.log1p(jnp.exp(-jnp.abs(p)))
        sn = jnp.maximum(n, 0.0) + jnp.log1p(jnp.exp(-jnp.abs(n)))
        out_ref[...] = ((jnp.sum(sp) + jnp.sum(sn)) * (1.0 / B)).reshape(1, 1)

    res = pl.pallas_call(
        body,
        out_shape=jax.ShapeDtypeStruct((1, 1), jnp.float32),
    )(pos.reshape(B // 128, 128), neg.reshape(B * NEG // 128, 128))
    return res[0, 0]


def kernel(x, targets, negatives, emb_table, out_weight):
    B = x.shape[0]
    x = x.astype(jnp.int32)
    targets = targets.astype(jnp.int32)
    neg_flat = negatives.astype(jnp.int32).reshape(-1)
    pos_s, neg_s = _sc_scores(x, targets, neg_flat, emb_table, out_weight)
    return _tc_loss(pos_s, neg_s, B)


# speedup vs baseline: 1.9124x; 1.3598x over previous
import functools

import jax
import jax.numpy as jnp
from jax import lax
from jax.experimental import pallas as pl
from jax.experimental.pallas import tpu as pltpu
from jax.experimental.pallas import tpu_sc as plsc

DIM = 64
NEG = 10
L = 16
C = 64


def _sc_scores(x, targets, neg_flat, emb_table, out_weight):
    B = x.shape[0]
    info = plsc.get_sparse_core_info()
    NC, NS = info.num_cores, info.num_subcores
    NW = NC * NS
    per_w = B // NW
    n_chunks = per_w // C

    mesh = plsc.VectorSubcoreMesh(core_axis_name="c", subcore_axis_name="s")

    @functools.partial(
        pl.kernel,
        mesh=mesh,
        compiler_params=pltpu.CompilerParams(
            needs_layout_passes=False, use_tc_tiling_on_sc=True
        ),
        out_type=(
            jax.ShapeDtypeStruct((B,), jnp.float32),
            jax.ShapeDtypeStruct((B * NEG,), jnp.float32),
        ),
        scratch_types=[
            pltpu.VMEM((C,), jnp.int32),
            pltpu.VMEM((C,), jnp.int32),
            pltpu.VMEM((C * NEG,), jnp.int32),
            pltpu.VMEM((C, DIM), jnp.float32),
            pltpu.VMEM((C, DIM), jnp.float32),
            pltpu.VMEM((C * NEG, DIM), jnp.float32),
            pltpu.VMEM((C,), jnp.float32),
            pltpu.VMEM((C * NEG,), jnp.float32),
            pltpu.SemaphoreType.DMA,
        ],
    )
    def k(x_h, t_h, n_h, emb3_h, w3_h, pos_h, negs_h,
          vi_x, vi_t, vi_n, emb_v, pos_v, neg_v,
          pos_o, neg_o, sem):
        emb_h = emb3_h.at[0]
        w_h = w3_h.at[0]
        wid = lax.axis_index("s") * NC + lax.axis_index("c")
        base = wid * per_w

        def chunk_body(ci, _):
            cbase = base + ci * C
            pltpu.sync_copy(x_h.at[pl.ds(cbase, C)], vi_x)
            pltpu.sync_copy(t_h.at[pl.ds(cbase, C)], vi_t)
            pltpu.sync_copy(n_h.at[pl.ds(cbase * NEG, C * NEG)], vi_n)

            def row_issue(g, _):
                vx = vi_x[pl.ds(g * L, L)]
                vt = vi_t[pl.ds(g * L, L)]
                for t in range(L):
                    i = g * L + t
                    pltpu.async_copy(emb_h.at[vx[t]], emb_v.at[i], sem)
                    pltpu.async_copy(w_h.at[vt[t]], pos_v.at[i], sem)
                return 0

            def neg_issue(g, _):
                vn = vi_n[pl.ds(g * L, L)]
                for t in range(L):
                    i = g * L + t
                    pltpu.async_copy(w_h.at[vn[t]], neg_v.at[i], sem)
                return 0

            lax.fori_loop(0, C // L, row_issue, 0)
            lax.fori_loop(0, C * NEG // L, neg_issue, 0)
            pltpu.make_async_copy(emb_h.at[pl.ds(0, C)], emb_v, sem).wait()
            pltpu.make_async_copy(w_h.at[pl.ds(0, C)], pos_v, sem).wait()
            pltpu.make_async_copy(w_h.at[pl.ds(0, C * NEG)], neg_v, sem).wait()

            def body(g, _):
                rows = g * L + lax.iota(jnp.int32, L)
                nrows = [rows * NEG + j for j in range(NEG)]
                pos_acc = jnp.zeros((L,), jnp.float32)
                neg_accs = [jnp.zeros((L,), jnp.float32) for _ in range(NEG)]
                for d in range(DIM):
                    cold = jnp.full((L,), d, jnp.int32)
                    ev = plsc.load_gather(emb_v, [rows, cold])
                    pv = plsc.load_gather(pos_v, [rows, cold])
                    pos_acc = pos_acc + ev * pv
                    for j in range(NEG):
                        nv = plsc.load_gather(neg_v, [nrows[j], cold])
                        neg_accs[j] = neg_accs[j] + ev * nv
                pos_o[pl.ds(g * L, L)] = pos_acc
                for j in range(NEG):
                    neg_o[pl.ds(j * C + g * L, L)] = neg_accs[j]
                return 0

            lax.fori_loop(0, C // L, body, 0)
            pltpu.sync_copy(pos_o, pos_h.at[pl.ds(cbase, C)])
            pltpu.sync_copy(neg_o, negs_h.at[pl.ds(cbase * NEG, C * NEG)])
            return 0

        lax.fori_loop(0, n_chunks, chunk_body, 0)

    V = emb_table.shape[0]
    return k(x, targets, neg_flat,
             emb_table.reshape(1, V, DIM), out_weight.reshape(1, V, DIM))


def _tc_loss(pos, neg, B):
    def body(pos_ref, neg_ref, out_ref):
        p = pos_ref[...]
        n = neg_ref[...]
        sp = jnp.maximum(-p, 0.0) + jnp.log1p(jnp.exp(-jnp.abs(p)))
        sn = jnp.maximum(n, 0.0) + jnp.log1p(jnp.exp(-jnp.abs(n)))
        out_ref[...] = ((jnp.sum(sp) + jnp.sum(sn)) * (1.0 / B)).reshape(1, 1)

    res = pl.pallas_call(
        body,
        out_shape=jax.ShapeDtypeStruct((1, 1), jnp.float32),
    )(pos.reshape(B // 128, 128), neg.reshape(B * NEG // 128, 128))
    return res[0, 0]


def kernel(x, targets, negatives, emb_table, out_weight):
    B = x.shape[0]
    x = x.astype(jnp.int32)
    targets = targets.astype(jnp.int32)
    neg_flat = negatives.astype(jnp.int32).reshape(-1)
    pos_s, neg_s = _sc_scores(x, targets, neg_flat, emb_table, out_weight)
    return _tc_loss(pos_s, neg_s, B)
